# baseline (speedup 1.0000x reference)
import jax
import jax.numpy as jnp
from jax.experimental import pallas as pl
from jax.experimental.pallas import tpu as pltpu

_BT = 512


def _fused_body(x_ref, a1_ref, a2_ref, sh1_ref, sh2_ref,
                wfc1_ref, bfc1_ref, wfc2_ref, bfc2_ref, o_ref):
    bt = x_ref.shape[0]
    bf = jnp.bfloat16

    gc = [x_ref[:, c] for c in range(3)]
    xh = [jnp.concatenate(
        [gc[c][:, h // 4, 32 * (h % 4):32 * (h % 4) + 32] for c in range(3)],
        axis=1) for h in range(32)]

    sh1 = sh1_ref[...]
    sh2 = sh2_ref[...]

    def conv1_row(h):
        lo, hi = max(h - 1, 0), min(h + 1, 31)
        x3 = (xh[lo] if lo == hi else
              jnp.concatenate(xh[lo:hi + 1], axis=1))
        k0, k1 = (lo - h + 1) * 96, (hi - h + 2) * 96
        return (jnp.dot(x3, a1_ref[k0:k1, 0:256],
                        preferred_element_type=jnp.float32),
                jnp.dot(x3, a1_ref[k0:k1, 256:512],
                        preferred_element_type=jnp.float32))

    y1 = []
    for k in range(16):
        e0, o0 = conv1_row(2 * k)
        e1, o1 = conv1_row(2 * k + 1)
        m = jnp.maximum(jnp.maximum(e0, o0), jnp.maximum(e1, o1))
        y1.append(jnp.maximum(m + sh1, 0.0).astype(bf))

    def conv2_row(h):
        acc_e = None
        acc_o = None
        for di in range(3):
            src = h + di - 1
            if src < 0 or src > 15:
                continue
            e = jnp.dot(y1[src], a2_ref[256 * di:256 * di + 256, 0:256],
                        preferred_element_type=jnp.float32)
            o = jnp.dot(y1[src], a2_ref[256 * di:256 * di + 256, 256:512],
                        preferred_element_type=jnp.float32)
            acc_e = e if acc_e is None else acc_e + e
            acc_o = o if acc_o is None else acc_o + o
        return acc_e, acc_o

    y2 = []
    for r in range(8):
        e0, o0 = conv2_row(2 * r)
        e1, o1 = conv2_row(2 * r + 1)
        m = jnp.maximum(jnp.maximum(e0, o0), jnp.maximum(e1, o1))
        y2.append(jnp.maximum(m + sh2, 0.0))

    flat = jnp.concatenate(y2, axis=1)

    h = jnp.maximum(
        jnp.dot(flat, wfc1_ref[...], preferred_element_type=jnp.float32)
        + bfc1_ref[...], 0.0)
    z = jnp.sum(h * wfc2_ref[...], axis=-1, keepdims=True) + bfc2_ref[...]
    o_ref[...] = 1.0 / (1.0 + jnp.exp(-z))


def kernel(x_nchw, a1e, a1o, a2e, a2o, sc1, sh1, sc2, sh2,
           wfc1, bfc1, wfc2, bfc2):
    n = x_nchw.shape[0]
    bt = min(_BT, n)
    bf = jnp.bfloat16

    x4 = x_nchw.reshape(n, 3, 8, 128).astype(bf)

    def stack1(a):
        return a.reshape(3, 32, 3, 256).transpose(0, 2, 1, 3).reshape(288, 256)

    a1 = (jnp.concatenate([stack1(a1e), stack1(a1o)], axis=1)
          * jnp.concatenate([sc1, sc1], axis=1)).astype(bf)
    a2 = (jnp.concatenate([a2e.reshape(768, 256), a2o.reshape(768, 256)],
                          axis=1)
          * jnp.concatenate([sc2, sc2], axis=1)).astype(bf)

    c2 = lambda i: (0, 0)
    out = pl.pallas_call(
        _fused_body,
        out_shape=jax.ShapeDtypeStruct((n, 1), jnp.float32),
        grid=(n // bt,),
        in_specs=[
            pl.BlockSpec((bt, 3, 8, 128), lambda i: (i, 0, 0, 0)),
            pl.BlockSpec((288, 512), c2),
            pl.BlockSpec((768, 512), c2),
            pl.BlockSpec((1, 256), c2),
            pl.BlockSpec((1, 256), c2),
            pl.BlockSpec((2048, 64), c2),
            pl.BlockSpec((1, 64), c2),
            pl.BlockSpec((1, 64), c2),
            pl.BlockSpec((1, 1), c2),
        ],
        out_specs=pl.BlockSpec((bt, 1), lambda i: (i, 0)),
        compiler_params=pltpu.CompilerParams(
            dimension_semantics=("arbitrary",)),
    )(x4, a1, a2, sh1, sh2, wfc1, bfc1, wfc2, bfc2)
    return out


# speedup vs baseline: 1.0422x; 1.0422x over previous
import jax
import jax.numpy as jnp
from jax.experimental import pallas as pl
from jax.experimental.pallas import tpu as pltpu

_BT = 512


def _fused_body(x_ref, a1_ref, a2_ref, sh1_ref, sh2_ref,
                wfc1_ref, bfc1_ref, wfc2_ref, bfc2_ref, o_ref):
    bf = jnp.bfloat16

    xh = [x_ref[h] for h in range(32)]

    sh1 = sh1_ref[...]
    sh2 = sh2_ref[...]

    def conv1_row(h):
        lo, hi = max(h - 1, 0), min(h + 1, 31)
        x3 = (xh[lo] if lo == hi else
              jnp.concatenate(xh[lo:hi + 1], axis=1))
        k0, k1 = (lo - h + 1) * 96, (hi - h + 2) * 96
        return jnp.dot(x3, a1_ref[k0:k1, :],
                       preferred_element_type=jnp.float32)

    y1 = []
    for k in range(16):
        a = conv1_row(2 * k)
        b = conv1_row(2 * k + 1)
        m = jnp.maximum(a, b)
        m = jnp.maximum(m[:, 0:256], m[:, 256:512])
        y1.append(jnp.maximum(m + sh1, 0.0).astype(bf))

    def conv2_row(h):
        acc = None
        for di in range(3):
            src = h + di - 1
            if src < 0 or src > 15:
                continue
            d = jnp.dot(y1[src], a2_ref[256 * di:256 * di + 256, :],
                        preferred_element_type=jnp.float32)
            acc = d if acc is None else acc + d
        return acc

    y2 = []
    for r in range(8):
        a = conv2_row(2 * r)
        b = conv2_row(2 * r + 1)
        m = jnp.maximum(a, b)
        m = jnp.maximum(m[:, 0:256], m[:, 256:512])
        y2.append(jnp.maximum(m + sh2, 0.0))

    flat = jnp.concatenate(y2, axis=1)

    h = jnp.maximum(
        jnp.dot(flat, wfc1_ref[...], preferred_element_type=jnp.float32)
        + bfc1_ref[...], 0.0)
    z = jnp.sum(h * wfc2_ref[...], axis=-1, keepdims=True) + bfc2_ref[...]
    o_ref[...] = 1.0 / (1.0 + jnp.exp(-z))


def kernel(x_nchw, a1e, a1o, a2e, a2o, sc1, sh1, sc2, sh2,
           wfc1, bfc1, wfc2, bfc2):
    n = x_nchw.shape[0]
    bt = min(_BT, n)
    bf = jnp.bfloat16

    x_t = jnp.transpose(x_nchw, (2, 0, 1, 3)).reshape(32, n, 96).astype(bf)

    def stack1(a):
        return a.reshape(3, 32, 3, 256).transpose(0, 2, 1, 3).reshape(288, 256)

    a1 = (jnp.concatenate([stack1(a1e), stack1(a1o)], axis=1)
          * jnp.concatenate([sc1, sc1], axis=1)).astype(bf)
    a2 = (jnp.concatenate([a2e.reshape(768, 256), a2o.reshape(768, 256)],
                          axis=1)
          * jnp.concatenate([sc2, sc2], axis=1)).astype(bf)

    c2 = lambda i: (0, 0)
    out = pl.pallas_call(
        _fused_body,
        out_shape=jax.ShapeDtypeStruct((n, 1), jnp.float32),
        grid=(n // bt,),
        in_specs=[
            pl.BlockSpec((32, bt, 96), lambda i: (0, i, 0)),
            pl.BlockSpec((288, 512), c2),
            pl.BlockSpec((768, 512), c2),
            pl.BlockSpec((1, 256), c2),
            pl.BlockSpec((1, 256), c2),
            pl.BlockSpec((2048, 64), c2),
            pl.BlockSpec((1, 64), c2),
            pl.BlockSpec((1, 64), c2),
            pl.BlockSpec((1, 1), c2),
        ],
        out_specs=pl.BlockSpec((bt, 1), lambda i: (i, 0)),
        compiler_params=pltpu.CompilerParams(
            dimension_semantics=("arbitrary",)),
    )(x_t, a1, a2, sh1, sh2, wfc1, bfc1, wfc2, bfc2)
    return out
